# baseline (speedup 1.0000x reference)
import functools

import jax
import jax.numpy as jnp
from jax import lax
from jax.experimental import pallas as pl
from jax.experimental.pallas import tpu as pltpu
from jax.experimental.pallas import tpu_sc as plsc

_H = 14
_W = 14
_HW = _H * _W


def _sc_gather(idx, table2):
    B = idx.shape[0]
    D2 = table2.shape[1]
    info = plsc.get_sparse_core_info()
    nw = info.num_cores * info.num_subcores
    b_per_w = B // nw
    mesh = plsc.VectorSubcoreMesh(core_axis_name="c", subcore_axis_name="s")

    @functools.partial(
        pl.kernel,
        mesh=mesh,
        out_type=jax.ShapeDtypeStruct((B, D2), jnp.float32),
        scratch_types=[
            pltpu.VMEM((b_per_w,), jnp.int32),
            pltpu.VMEM((b_per_w, D2), jnp.float32),
            pltpu.SemaphoreType.DMA,
        ],
    )
    def k(idx_hbm, table_hbm, out_hbm, idx_v, rows_v, sem):
        wid = lax.axis_index("s") * info.num_cores + lax.axis_index("c")
        base = wid * b_per_w
        pltpu.sync_copy(idx_hbm.at[pl.ds(base, b_per_w)], idx_v)
        pltpu.async_copy(table_hbm.at[idx_v], rows_v, sem).wait()
        pltpu.sync_copy(rows_v, out_hbm.at[pl.ds(base, b_per_w)])

    return k(idx, table2)


def _tc_select_broadcast(wide, parity, D):
    B = wide.shape[0]
    bb = 128

    def body(w_ref, p_ref, o_ref):
        w = w_ref[...]
        par = p_ref[...] > 0
        sel = jnp.where(par, w[:, D:], w[:, :D])
        o_ref[...] = jnp.broadcast_to(sel[:, :, None], (bb, D, _HW))

    return pl.pallas_call(
        body,
        grid=(B // bb,),
        in_specs=[
            pl.BlockSpec((bb, 2 * D), lambda i: (i, 0)),
            pl.BlockSpec((bb, 1), lambda i: (i, 0)),
        ],
        out_specs=pl.BlockSpec((bb, D, _HW), lambda i: (i, 0, 0)),
        out_shape=jax.ShapeDtypeStruct((B, D, _HW), jnp.float32),
    )(wide, parity)


def kernel(labels, table):
    B = labels.shape[0]
    V, D = table.shape
    labels = labels.astype(jnp.int32)
    table2 = table.reshape(V // 2, 2 * D)
    wide = _sc_gather(labels >> 1, table2)
    parity = (labels & 1).reshape(B, 1)
    out = _tc_select_broadcast(wide, parity, D)
    return out.reshape(B, D, _H, _W)
